# baseline (speedup 1.0000x reference)
import functools

import numpy as np

import jax
import jax.numpy as jnp
from jax import lax
from jax.experimental import pallas as pl
from jax.experimental.pallas import tpu as pltpu
from jax.experimental.pallas import tpu_sc as plsc

B = 16384
D = 128
NC = 2
NS = 16
L = 16
NW = NC * NS
BPW = B // NW
NG = BPW // L
R = 2
SKEW = 17
def _mf_body(uid_hbm, iid_hbm, ue_hbm, ie_hbm, ub_hbm, ib_hbm, out_hbm,
             uidx_v, iidx_v, ue_r, ie_r, ub_r, ib_r, out_v, pscr_v, *sems):
    wid = lax.axis_index("s") * NC + lax.axis_index("c")
    base = wid * BPW

    pltpu.sync_copy(uid_hbm.at[pl.ds(base, BPW)], uidx_v)
    pltpu.sync_copy(iid_hbm.at[pl.ds(base, BPW)], iidx_v)

    lane = lax.iota(jnp.int32, L)
    lane_skew = lane * SKEW
    tcol = [lane_skew + j for j in range(L)]

    def issue(g, slot):
        uix = uidx_v[pl.ds(g * L, L)]
        iix = iidx_v[pl.ds(g * L, L)]
        pltpu.async_copy(ue_hbm.at[uix], ue_r.at[slot], sems[slot])
        pltpu.async_copy(ie_hbm.at[iix], ie_r.at[slot], sems[slot])
        pltpu.async_copy(ub_hbm.at[uix], ub_r.at[slot], sems[slot])
        pltpu.async_copy(ib_hbm.at[iix], ib_r.at[slot], sems[slot])

    for b in range(R):
        issue(jnp.int32(b), b)

    def outer(i, carry):
        g0 = i * R
        for b in range(R):
            g = g0 + b
            pltpu.make_async_copy(
                ue_hbm.at[pl.ds(0, L)], ue_r.at[b], sems[b]).wait()
            pltpu.make_async_copy(
                ie_hbm.at[pl.ds(0, L)], ie_r.at[b], sems[b]).wait()
            pltpu.make_async_copy(
                ub_hbm.at[pl.ds(0, L)], ub_r.at[b], sems[b]).wait()
            pltpu.make_async_copy(
                ib_hbm.at[pl.ds(0, L)], ib_r.at[b], sems[b]).wait()

            ue_s = ue_r.at[b]
            ie_s = ie_r.at[b]

            def ex_body(e, carry):
                for s in range(2):
                    ee = e * 2 + s
                    ts = []
                    for k in range(D // L):
                        u = ue_s[ee, pl.ds(k * L, L)]
                        v = ie_s[ee, pl.ds(k * L, L)]
                        ts.append(u * v)
                    while len(ts) > 1:
                        ts = [a + c for a, c in
                              zip(ts[::2], ts[1::2])]
                    pscr_v[pl.ds(ee * SKEW, L)] = ts[0]
                return carry

            lax.fori_loop(0, L // 2, ex_body, 0)

            gs = [plsc.load_gather(pscr_v, [tcol[j]]) for j in range(L)]
            while len(gs) > 1:
                gs = [a + c for a, c in zip(gs[::2], gs[1::2])]
            acc = gs[0] + ub_r[b, :] + ib_r[b, :]
            out_v[pl.ds(g * L, L)] = 1.0 / (1.0 + jnp.exp(-acc))

            @pl.when(g + R < NG)
            def _():
                issue(g + R, b)
        return carry

    lax.fori_loop(0, NG // R, outer, 0)
    pltpu.sync_copy(out_v, out_hbm.at[pl.ds(base, BPW)])


@jax.jit
def _mf_sc(uid, iid, ue, ie, ub, ib):
    mesh = plsc.VectorSubcoreMesh(core_axis_name="c", subcore_axis_name="s")
    fn = functools.partial(
        pl.kernel,
        mesh=mesh,
        out_type=jax.ShapeDtypeStruct((B,), jnp.float32),
        scratch_types=[
            pltpu.VMEM((BPW,), jnp.int32),
            pltpu.VMEM((BPW,), jnp.int32),
            pltpu.VMEM((R, L, D), jnp.float32),
            pltpu.VMEM((R, L, D), jnp.float32),
            pltpu.VMEM((R, L), jnp.float32),
            pltpu.VMEM((R, L), jnp.float32),
            pltpu.VMEM((BPW,), jnp.float32),
            pltpu.VMEM((L * SKEW,), jnp.float32),
        ] + [pltpu.SemaphoreType.DMA] * R,
        compiler_params=pltpu.CompilerParams(needs_layout_passes=False),
    )(_mf_body)
    return fn(uid, iid, ue, ie, ub, ib)


def kernel(user_ids, item_ids, dummy_a, dummy_b, user_emb, item_emb,
           user_bias, item_bias):
    uid = user_ids.astype(jnp.int32)
    iid = item_ids.astype(jnp.int32)
    return _mf_sc(uid, iid, user_emb, item_emb,
                  user_bias.reshape(-1), item_bias.reshape(-1))


# speedup vs baseline: 1.1920x; 1.1920x over previous
import functools

import jax
import jax.numpy as jnp
from jax import lax
from jax.experimental import pallas as pl
from jax.experimental.pallas import tpu as pltpu
from jax.experimental.pallas import tpu_sc as plsc

B = 16384
D = 128
NC = 2
NS = 16
L = 16
NW = NC * NS
BPW = B // NW
NG = BPW // L
R = 4
SKEW = 17
def _mf_body(uid_hbm, iid_hbm, ue_hbm, ie_hbm, ub_hbm, ib_hbm, out_hbm,
             uidx_v, iidx_v, ue_r, ie_r, ub_r, ib_r, out_v, pscr_v, *sems):
    wid = lax.axis_index("s") * NC + lax.axis_index("c")
    base = wid * BPW

    pltpu.sync_copy(uid_hbm.at[pl.ds(base, BPW)], uidx_v)
    pltpu.sync_copy(iid_hbm.at[pl.ds(base, BPW)], iidx_v)

    lane = lax.iota(jnp.int32, L)
    lane_skew = lane * SKEW
    tcol = [lane_skew + j for j in range(L)]

    def issue(g, slot):
        uix = uidx_v[pl.ds(g * L, L)]
        iix = iidx_v[pl.ds(g * L, L)]
        pltpu.async_copy(ue_hbm.at[uix], ue_r.at[slot], sems[slot])
        pltpu.async_copy(ie_hbm.at[iix], ie_r.at[slot], sems[slot])
        pltpu.async_copy(ub_hbm.at[uix], ub_r.at[slot], sems[slot])
        pltpu.async_copy(ib_hbm.at[iix], ib_r.at[slot], sems[slot])

    for b in range(R):
        issue(jnp.int32(b), b)

    def outer(i, carry):
        g0 = i * R
        for b in range(R):
            g = g0 + b
            pltpu.make_async_copy(
                ue_hbm.at[pl.ds(0, L)], ue_r.at[b], sems[b]).wait()
            pltpu.make_async_copy(
                ie_hbm.at[pl.ds(0, L)], ie_r.at[b], sems[b]).wait()
            pltpu.make_async_copy(
                ub_hbm.at[pl.ds(0, L)], ub_r.at[b], sems[b]).wait()
            pltpu.make_async_copy(
                ib_hbm.at[pl.ds(0, L)], ib_r.at[b], sems[b]).wait()

            ue_s = ue_r.at[b]
            ie_s = ie_r.at[b]

            def ex_body(e, carry):
                for s in range(2):
                    ee = e * 2 + s
                    ts = []
                    for k in range(D // L):
                        u = ue_s[ee, pl.ds(k * L, L)]
                        v = ie_s[ee, pl.ds(k * L, L)]
                        ts.append(u * v)
                    while len(ts) > 1:
                        ts = [a + c for a, c in
                              zip(ts[::2], ts[1::2])]
                    pscr_v[pl.ds(ee * SKEW, L)] = ts[0]
                return carry

            lax.fori_loop(0, L // 2, ex_body, 0)

            gs = [plsc.load_gather(pscr_v, [tcol[j]]) for j in range(L)]
            while len(gs) > 1:
                gs = [a + c for a, c in zip(gs[::2], gs[1::2])]
            acc = gs[0] + ub_r[b, :] + ib_r[b, :]
            out_v[pl.ds(g * L, L)] = 1.0 / (1.0 + jnp.exp(-acc))

            @pl.when(g + R < NG)
            def _():
                issue(g + R, b)
        return carry

    lax.fori_loop(0, NG // R, outer, 0)
    pltpu.sync_copy(out_v, out_hbm.at[pl.ds(base, BPW)])


@jax.jit
def _mf_sc(uid, iid, ue, ie, ub, ib):
    mesh = plsc.VectorSubcoreMesh(core_axis_name="c", subcore_axis_name="s")
    fn = functools.partial(
        pl.kernel,
        mesh=mesh,
        out_type=jax.ShapeDtypeStruct((B,), jnp.float32),
        scratch_types=[
            pltpu.VMEM((BPW,), jnp.int32),
            pltpu.VMEM((BPW,), jnp.int32),
            pltpu.VMEM((R, L, D), jnp.float32),
            pltpu.VMEM((R, L, D), jnp.float32),
            pltpu.VMEM((R, L), jnp.float32),
            pltpu.VMEM((R, L), jnp.float32),
            pltpu.VMEM((BPW,), jnp.float32),
            pltpu.VMEM((L * SKEW,), jnp.float32),
        ] + [pltpu.SemaphoreType.DMA] * R,
        compiler_params=pltpu.CompilerParams(needs_layout_passes=False),
    )(_mf_body)
    return fn(uid, iid, ue, ie, ub, ib)


def kernel(user_ids, item_ids, dummy_a, dummy_b, user_emb, item_emb,
           user_bias, item_bias):
    uid = user_ids.astype(jnp.int32)
    iid = item_ids.astype(jnp.int32)
    return _mf_sc(uid, iid, user_emb, item_emb,
                  user_bias.reshape(-1), item_bias.reshape(-1))
